# baseline (speedup 1.0000x reference)
import functools

import jax
import jax.numpy as jnp
from jax import lax
from jax.experimental import pallas as pl
from jax.experimental.pallas import tpu as pltpu
from jax.experimental.pallas import tpu_sc as plsc

VOCAB = 100000
D = 64
B = 16384
NC, NS = 2, 16
NW = NC * NS
BPW = B // NW
CHUNK = 128
NCHUNK = BPW // CHUNK

@functools.cache
def _build_sc_gather():
    mesh = plsc.VectorSubcoreMesh(core_axis_name="c", subcore_axis_name="s")

    @functools.partial(
        pl.kernel,
        mesh=mesh,
        out_type=(
            jax.ShapeDtypeStruct((B, D), jnp.float32),
            jax.ShapeDtypeStruct((B, D), jnp.float32),
        ),
        scratch_types=[
            pltpu.VMEM((NCHUNK, CHUNK), jnp.int32),
            pltpu.VMEM((NCHUNK, CHUNK), jnp.int32),
            pltpu.VMEM((BPW, D), jnp.float32),
            pltpu.VMEM((BPW, D), jnp.float32),
            pltpu.SemaphoreType.DMA,
        ],
        compiler_params=pltpu.CompilerParams(use_tc_tiling_on_sc=False),
    )
    def _sc_gather(ctab, xtab, ci, xi, a_out, b_out, ci_v, xi_v, a_v, b_v, sem):
        wid = lax.axis_index("s") * NC + lax.axis_index("c")
        pltpu.sync_copy(ci.at[pl.ds(wid * NCHUNK, NCHUNK)], ci_v)
        pltpu.sync_copy(xi.at[pl.ds(wid * NCHUNK, NCHUNK)], xi_v)
        copies = []
        for j in range(NCHUNK):
            copies.append(
                pltpu.async_copy(ctab.at[ci_v.at[j]], a_v.at[pl.ds(j * CHUNK, CHUNK)], sem)
            )
            copies.append(
                pltpu.async_copy(xtab.at[xi_v.at[j]], b_v.at[pl.ds(j * CHUNK, CHUNK)], sem)
            )
        for c in copies:
            c.wait()
        base = wid * BPW
        pltpu.sync_copy(a_v, a_out.at[pl.ds(base, BPW)])
        pltpu.sync_copy(b_v, b_out.at[pl.ds(base, BPW)])

    return _sc_gather


def _prob_body(a_ref, b_ref, o_ref):
    a = a_ref[...]
    b = b_ref[...]
    dot = jnp.sum(a * b, axis=1, keepdims=True)
    na = jnp.sqrt(jnp.sum(a * a, axis=1, keepdims=True))
    nb = jnp.sqrt(jnp.sum(b * b, axis=1, keepdims=True))
    denom = jnp.maximum(na * nb, 1e-8)
    o_ref[...] = (1.0 + dot / denom) * 0.5


ROWS_BLK = 2048
_prob = pl.pallas_call(
    _prob_body,
    grid=(B // ROWS_BLK,),
    in_specs=[
        pl.BlockSpec((ROWS_BLK, D), lambda i: (i, 0)),
        pl.BlockSpec((ROWS_BLK, D), lambda i: (i, 0)),
    ],
    out_specs=pl.BlockSpec((ROWS_BLK, 1), lambda i: (i, 0)),
    out_shape=jax.ShapeDtypeStruct((B, 1), jnp.float32),
)


def kernel(center_table, context_table, center, context):
    ci = center.astype(jnp.int32).reshape(NW * NCHUNK, CHUNK)
    xi = context.astype(jnp.int32).reshape(NW * NCHUNK, CHUNK)
    a, b = _build_sc_gather()(center_table, context_table, ci, xi)
    return _prob(a, b).reshape(B)
